# baseline (speedup 1.0000x reference)
import jax
import jax.numpy as jnp
from jax.experimental import pallas as pl
from jax.experimental.pallas import tpu as pltpu
from jax.experimental.pallas import tpu_sc as plsc


def _sc_gather(table, ids, n, embed):
    b, s = ids.shape
    rows_per_block = 2
    mesh = plsc.VectorSubcoreMesh(core_axis_name="c", subcore_axis_name="s")

    @pl.kernel(
        out_type=jax.ShapeDtypeStruct((n, embed), jnp.float32),
        mesh=mesh,
        compiler_params=pltpu.CompilerParams(use_tc_tiling_on_sc=False),
    )
    def gather_kernel(table_hbm, ids_hbm, out_hbm):
        def body(i_vmem, o_vmem):
            for r in range(rows_per_block):
                pltpu.sync_copy(
                    table_hbm.at[i_vmem.at[r]],
                    o_vmem.at[pl.ds(r * s, s)],
                )

        pltpu.emit_pipeline(
            body,
            grid=(b // rows_per_block,),
            in_specs=[pl.BlockSpec((rows_per_block, s), lambda i: (i, 0))],
            out_specs=[pl.BlockSpec((rows_per_block * s, embed), lambda i: (i, 0))],
            core_axis_name=("c", "s"),
            dimension_semantics=(pltpu.PARALLEL,),
        )(ids_hbm, out_hbm)

    return gather_kernel(table, ids)


def _tc_body(g_ref, m_ref, p_ref, me_ref, ga_ref, be_ref, o_ref):
    bb, s, embed = o_ref.shape
    x = g_ref[...]
    m = m_ref[...]
    x = x * (1.0 - m) + me_ref[...] * m
    x = x + jnp.tile(p_ref[...], (bb, 1))
    ones = jnp.ones((embed, embed), dtype=jnp.float32)
    mean = jax.lax.dot(x, ones, preferred_element_type=jnp.float32) * (1.0 / embed)
    meansq = jax.lax.dot(x * x, ones, preferred_element_type=jnp.float32) * (1.0 / embed)
    var = meansq - mean * mean
    y = (x - mean) * jax.lax.rsqrt(var + 1e-5) * ga_ref[...] + be_ref[...]
    o_ref[...] = y.reshape(bb, s, embed)


def kernel(input_ids, mask, token_table, pos_table, mask_embedding, gamma, beta):
    b, s = input_ids.shape
    vocab, embed = token_table.shape
    n = b * s

    gathered = _sc_gather(token_table, input_ids.astype(jnp.int32), n, embed)

    mask_f = mask.astype(jnp.float32).reshape(n, 1)
    pos = pos_table[:s]
    me = mask_embedding.reshape(1, embed)
    ga = gamma.reshape(1, embed)
    be = beta.reshape(1, embed)

    bb = 8
    grid = (b // bb,)
    out = pl.pallas_call(
        _tc_body,
        grid=grid,
        in_specs=[
            pl.BlockSpec((bb * s, embed), lambda i: (i, 0)),
            pl.BlockSpec((bb * s, 1), lambda i: (i, 0)),
            pl.BlockSpec((s, embed), lambda i: (0, 0)),
            pl.BlockSpec((1, embed), lambda i: (0, 0)),
            pl.BlockSpec((1, embed), lambda i: (0, 0)),
            pl.BlockSpec((1, embed), lambda i: (0, 0)),
        ],
        out_specs=pl.BlockSpec((bb, s, embed), lambda i: (i, 0, 0)),
        out_shape=jax.ShapeDtypeStruct((b, s, embed), jnp.float32),
    )(gathered, mask_f, pos, me, ga, be)
    return out
